# Initial kernel scaffold; baseline (speedup 1.0000x reference)
import functools

import jax
import jax.numpy as jnp
from jax import lax
from jax.experimental import pallas as pl
from jax.experimental.pallas import tpu as pltpu
from jax.experimental.pallas import tpu_sc as plsc

N = 10000
NPAD = 10240
E = 320000
CH = 128
NCHUNK = 2560
EPAD = NCHUNK * CH
NSUB = 16
NCORE = 2
CPW = NCHUNK // (NSUB * NCORE)
CPS = NCHUNK // NSUB
STRIPE = NPAD // NSUB

_mesh = plsc.VectorSubcoreMesh(core_axis_name="c", subcore_axis_name="s")


@functools.partial(
    pl.kernel,
    out_type=jax.ShapeDtypeStruct((NCORE, NPAD), jnp.float32),
    mesh=_mesh,
    scratch_types=[
        pltpu.VMEM((CPW, CH), jnp.int32),
        pltpu.VMEM((CH,), jnp.float32),
        pltpu.VMEM((STRIPE,), jnp.float32),
        pltpu.VMEM_SHARED((NPAD,), jnp.float32),
    ],
)
def _deg_kernel(dst_hbm, deg_hbm, idx_v, ones_v, zeros_v, deg_sh):
    c = lax.axis_index("c")
    s = lax.axis_index("s")
    w = c * NSUB + s

    @pl.loop(0, CH, step=16)
    def _(i):
        ones_v[pl.ds(i, 16)] = jnp.ones((16,), jnp.float32)

    @pl.loop(0, STRIPE, step=16)
    def _(i):
        zeros_v[pl.ds(i, 16)] = jnp.zeros((16,), jnp.float32)

    pltpu.sync_copy(zeros_v, deg_sh.at[pl.ds(s * STRIPE, STRIPE)])
    pltpu.sync_copy(dst_hbm.at[pl.ds(w * CPW, CPW)], idx_v)
    plsc.subcore_barrier()

    @pl.loop(0, CPW)
    def _(j):
        pltpu.sync_copy(ones_v, deg_sh.at[idx_v.at[j]], add=True)

    plsc.subcore_barrier()
    pltpu.sync_copy(deg_sh.at[pl.ds(s * STRIPE, STRIPE)],
                    deg_hbm.at[c, pl.ds(s * STRIPE, STRIPE)])


def _make_propagate(F):
    @functools.partial(
        pl.kernel,
        out_type=jax.ShapeDtypeStruct((NCORE, NPAD, F), jnp.float32),
        mesh=_mesh,
        scratch_types=[
            pltpu.VMEM((CPS, CH), jnp.int32),
            pltpu.VMEM((CPS, CH), jnp.int32),
            pltpu.VMEM((CH, F), jnp.float32),
            pltpu.VMEM_SHARED((NPAD, F), jnp.float32),
        ],
    )
    def _prop(table_hbm, src_hbm, dst_hbm, z_hbm, src_v, dst_v, rows_v, z_sh):
        c = lax.axis_index("c")
        s = lax.axis_index("s")
        pltpu.sync_copy(src_hbm.at[s], src_v)
        pltpu.sync_copy(dst_hbm.at[s], dst_v)
        pltpu.sync_copy(table_hbm.at[c, pl.ds(s * STRIPE, STRIPE)],
                        z_sh.at[pl.ds(s * STRIPE, STRIPE)])
        plsc.subcore_barrier()

        @pl.loop(0, CPS)
        def _(j):
            pltpu.sync_copy(table_hbm.at[c].at[src_v.at[j]], rows_v)
            pltpu.sync_copy(rows_v, z_sh.at[dst_v.at[j]], add=True)

        plsc.subcore_barrier()
        pltpu.sync_copy(z_sh.at[pl.ds(s * STRIPE, STRIPE)],
                        z_hbm.at[c, pl.ds(s * STRIPE, STRIPE)])

    return _prop


_prop128 = _make_propagate(128)
_prop64 = _make_propagate(64)


_BM = 1280


def _dot(a, b):
    return jax.lax.dot_general(a, b, (((1,), (0,)), ((), ())),
                               precision=lax.Precision.HIGHEST,
                               preferred_element_type=jnp.float32)


def _matmul(x, w):
    m, k = x.shape
    _, n = w.shape

    def body(x_ref, w_ref, o_ref):
        o_ref[...] = _dot(x_ref[...], w_ref[...])

    return pl.pallas_call(
        body,
        grid=(m // _BM,),
        in_specs=[pl.BlockSpec((_BM, k), lambda i: (i, 0)),
                  pl.BlockSpec((k, n), lambda i: (0, 0))],
        out_specs=pl.BlockSpec((_BM, n), lambda i: (i, 0)),
        out_shape=jax.ShapeDtypeStruct((m, n), jnp.float32),
    )(x, w)


def _dinv_of(deg_ref):
    d = deg_ref[...]
    return lax.rsqrt(1.0 + d[0] + d[1])[:, None]


def _scale_halves(xw, deg2):
    def body(xw_ref, deg_ref, o_ref):
        dinv = _dinv_of(deg_ref)
        xwb = xw_ref[...]
        o_ref[0] = xwb[:, :128] * dinv
        o_ref[1] = xwb[:, 128:] * dinv

    return pl.pallas_call(
        body,
        grid=(NPAD // _BM,),
        in_specs=[pl.BlockSpec((_BM, 256), lambda i: (i, 0)),
                  pl.BlockSpec((2, _BM), lambda i: (0, i))],
        out_specs=pl.BlockSpec((2, _BM, 128), lambda i: (0, i, 0)),
        out_shape=jax.ShapeDtypeStruct((2, NPAD, 128), jnp.float32),
    )(xw, deg2)


def _layer2(z1, deg2, b1, w2):
    def body(z_ref, deg_ref, b1_ref, w2_ref, o_ref):
        dinv = _dinv_of(deg_ref)
        b = b1_ref[...]
        h0 = jnp.maximum(z_ref[0] * dinv + b[None, :128], 0.0)
        h1 = jnp.maximum(z_ref[1] * dinv + b[None, 128:], 0.0)
        y = _dot(h0, w2_ref[:128]) + _dot(h1, w2_ref[128:])
        ys = y * dinv
        o_ref[0] = ys[:, :64]
        o_ref[1] = ys[:, 64:]

    return pl.pallas_call(
        body,
        grid=(NPAD // _BM,),
        in_specs=[pl.BlockSpec((2, _BM, 128), lambda i: (0, i, 0)),
                  pl.BlockSpec((2, _BM), lambda i: (0, i)),
                  pl.BlockSpec((256,), lambda i: (0,)),
                  pl.BlockSpec((256, 128), lambda i: (0, 0))],
        out_specs=pl.BlockSpec((2, _BM, 64), lambda i: (0, i, 0)),
        out_shape=jax.ShapeDtypeStruct((2, NPAD, 64), jnp.float32),
    )(z1, deg2, b1, w2)


def _finalize(z2, deg2, b2):
    def body(z_ref, deg_ref, b2_ref, o_ref):
        dinv = _dinv_of(deg_ref)
        b = b2_ref[...]
        o_ref[...] = jnp.concatenate(
            [z_ref[0] * dinv, z_ref[1] * dinv], axis=1) + b[None, :]

    return pl.pallas_call(
        body,
        grid=(NPAD // _BM,),
        in_specs=[pl.BlockSpec((2, _BM, 64), lambda i: (0, i, 0)),
                  pl.BlockSpec((2, _BM), lambda i: (0, i)),
                  pl.BlockSpec((128,), lambda i: (0,))],
        out_specs=pl.BlockSpec((_BM, 128), lambda i: (i, 0)),
        out_shape=jax.ShapeDtypeStruct((NPAD, 128), jnp.float32),
    )(z2, deg2, b2)


def kernel(x, adj_t, W1, b1, W2, b2):
    src = adj_t[0].astype(jnp.int32)
    dst = adj_t[1].astype(jnp.int32)
    pad = N + (jnp.arange(EPAD - E, dtype=jnp.int32) % (NPAD - N))
    srcp = jnp.concatenate([src, pad]).reshape(NCHUNK, CH)
    dstp = jnp.concatenate([dst, pad]).reshape(NCHUNK, CH)
    srcp3 = srcp.reshape(NSUB, CPS, CH)
    dstp3 = dstp.reshape(NSUB, CPS, CH)
    x_pad = jnp.pad(x, ((0, NPAD - N), (0, 0)))

    deg2 = _deg_kernel(dstp)
    xw1 = _matmul(x_pad, W1)
    xws1 = _scale_halves(xw1, deg2)
    z1 = _prop128(xws1, srcp3, dstp3)
    xws2 = _layer2(z1, deg2, b1, W2)
    z2 = _prop64(xws2, srcp3, dstp3)
    out = _finalize(z2, deg2, b2)
    return out[:N]


# speedup vs baseline: 18.3118x; 18.3118x over previous
import functools

import jax
import jax.numpy as jnp
from jax import lax
from jax.experimental import pallas as pl
from jax.experimental.pallas import tpu as pltpu
from jax.experimental.pallas import tpu_sc as plsc

N = 10000
NPAD = 10240
E = 320000
CH = 128
NCHUNK = 2560
EPAD = NCHUNK * CH
NSUB = 16
NCORE = 2
CPW = NCHUNK // (NSUB * NCORE)
CPS1 = NCHUNK // NSUB
CPS2 = NCHUNK // (NSUB * NCORE)
G = 40
STRIPE = NPAD // NSUB

_mesh = plsc.VectorSubcoreMesh(core_axis_name="c", subcore_axis_name="s")


@functools.partial(
    pl.kernel,
    out_type=jax.ShapeDtypeStruct((NCORE, NPAD), jnp.float32),
    mesh=_mesh,
    scratch_types=[
        pltpu.VMEM((CPW, CH), jnp.int32),
        pltpu.VMEM((CH,), jnp.float32),
        pltpu.VMEM((STRIPE,), jnp.float32),
        pltpu.VMEM_SHARED((NPAD,), jnp.float32),
    ],
)
def _deg_kernel(dst_hbm, deg_hbm, idx_v, ones_v, zeros_v, deg_sh):
    c = lax.axis_index("c")
    s = lax.axis_index("s")
    w = c * NSUB + s

    @pl.loop(0, CH, step=16)
    def _(i):
        ones_v[pl.ds(i, 16)] = jnp.ones((16,), jnp.float32)

    @pl.loop(0, STRIPE, step=16)
    def _(i):
        zeros_v[pl.ds(i, 16)] = jnp.zeros((16,), jnp.float32)

    pltpu.sync_copy(zeros_v, deg_sh.at[pl.ds(s * STRIPE, STRIPE)])
    pltpu.sync_copy(dst_hbm.at[pl.ds(w * CPW, CPW)], idx_v)
    plsc.subcore_barrier()

    @pl.loop(0, CPW)
    def _(j):
        pltpu.sync_copy(ones_v, deg_sh.at[idx_v.at[j]], add=True)

    plsc.subcore_barrier()
    pltpu.sync_copy(deg_sh.at[pl.ds(s * STRIPE, STRIPE)],
                    deg_hbm.at[c, pl.ds(s * STRIPE, STRIPE)])


@functools.partial(
    pl.kernel,
    out_type=jax.ShapeDtypeStruct((NCORE, NPAD, 128), jnp.float32),
    mesh=_mesh,
    scratch_types=[
        pltpu.VMEM((G, CH), jnp.int32),
        pltpu.VMEM((G, CH), jnp.int32),
        pltpu.VMEM((CH, 128), jnp.float32),
        pltpu.VMEM_SHARED((NPAD, 128), jnp.float32),
    ],
)
def _prop1(table_hbm, src_hbm, dst_hbm, z_hbm, src_v, dst_v, rows_v, z_sh):
    c = lax.axis_index("c")
    s = lax.axis_index("s")
    pltpu.sync_copy(table_hbm.at[c, pl.ds(s * STRIPE, STRIPE)],
                    z_sh.at[pl.ds(s * STRIPE, STRIPE)])
    plsc.subcore_barrier()

    @pl.loop(0, CPS1, step=G)
    def _(g):
        pltpu.sync_copy(src_hbm.at[s, pl.ds(g, G)], src_v)
        pltpu.sync_copy(dst_hbm.at[s, pl.ds(g, G)], dst_v)

        @pl.loop(0, G)
        def _(j):
            pltpu.sync_copy(table_hbm.at[c].at[src_v.at[j]], rows_v)
            pltpu.sync_copy(rows_v, z_sh.at[dst_v.at[j]], add=True)

    plsc.subcore_barrier()
    pltpu.sync_copy(z_sh.at[pl.ds(s * STRIPE, STRIPE)],
                    z_hbm.at[c, pl.ds(s * STRIPE, STRIPE)])


@functools.partial(
    pl.kernel,
    out_type=jax.ShapeDtypeStruct((NCORE, NPAD, 128), jnp.float32),
    mesh=_mesh,
    scratch_types=[
        pltpu.VMEM((G, CH), jnp.int32),
        pltpu.VMEM((G, CH), jnp.int32),
        pltpu.VMEM((CH, 128), jnp.float32),
        pltpu.VMEM_SHARED((NPAD, 128), jnp.float32),
    ],
)
def _prop2(table_hbm, src_hbm, dst_hbm, z_hbm, src_v, dst_v, rows_v, z_sh):
    c = lax.axis_index("c")
    s = lax.axis_index("s")
    pltpu.sync_copy(table_hbm.at[pl.ds(s * STRIPE, STRIPE)],
                    z_sh.at[pl.ds(s * STRIPE, STRIPE)])
    plsc.subcore_barrier()

    @pl.loop(0, CPS2, step=G)
    def _(g):
        pltpu.sync_copy(src_hbm.at[c, s, pl.ds(g, G)], src_v)
        pltpu.sync_copy(dst_hbm.at[c, s, pl.ds(g, G)], dst_v)

        @pl.loop(0, G)
        def _(j):
            pltpu.sync_copy(table_hbm.at[src_v.at[j]], rows_v)
            pltpu.sync_copy(rows_v, z_sh.at[dst_v.at[j]], add=True)

    plsc.subcore_barrier()
    pltpu.sync_copy(z_sh.at[pl.ds(s * STRIPE, STRIPE)],
                    z_hbm.at[c, pl.ds(s * STRIPE, STRIPE)])


_BM = 1280


def _dot(a, b):
    return jax.lax.dot_general(a, b, (((1,), (0,)), ((), ())),
                               precision=lax.Precision.HIGHEST,
                               preferred_element_type=jnp.float32)


def _matmul(x, w):
    m, k = x.shape
    _, n = w.shape

    def body(x_ref, w_ref, o_ref):
        o_ref[...] = _dot(x_ref[...], w_ref[...])

    return pl.pallas_call(
        body,
        grid=(m // _BM,),
        in_specs=[pl.BlockSpec((_BM, k), lambda i: (i, 0)),
                  pl.BlockSpec((k, n), lambda i: (0, 0))],
        out_specs=pl.BlockSpec((_BM, n), lambda i: (i, 0)),
        out_shape=jax.ShapeDtypeStruct((m, n), jnp.float32),
    )(x, w)


def _dinv_of(deg_ref):
    d = deg_ref[...]
    return lax.rsqrt(1.0 + d[0] + d[1])[:, None]


def _scale_halves(xw, deg2):
    def body(xw_ref, deg_ref, o_ref):
        dinv = _dinv_of(deg_ref)
        xwb = xw_ref[...]
        o_ref[0] = xwb[:, :128] * dinv
        o_ref[1] = xwb[:, 128:] * dinv

    return pl.pallas_call(
        body,
        grid=(NPAD // _BM,),
        in_specs=[pl.BlockSpec((_BM, 256), lambda i: (i, 0)),
                  pl.BlockSpec((2, _BM), lambda i: (0, i))],
        out_specs=pl.BlockSpec((2, _BM, 128), lambda i: (0, i, 0)),
        out_shape=jax.ShapeDtypeStruct((2, NPAD, 128), jnp.float32),
    )(xw, deg2)


def _layer2(z1, deg2, b1, w2):
    def body(z_ref, deg_ref, b1_ref, w2_ref, o_ref):
        dinv = _dinv_of(deg_ref)
        b = b1_ref[...]
        h0 = jnp.maximum(z_ref[0] * dinv + b[None, :128], 0.0)
        h1 = jnp.maximum(z_ref[1] * dinv + b[None, 128:], 0.0)
        y = _dot(h0, w2_ref[:128]) + _dot(h1, w2_ref[128:])
        o_ref[...] = y * dinv

    return pl.pallas_call(
        body,
        grid=(NPAD // _BM,),
        in_specs=[pl.BlockSpec((2, _BM, 128), lambda i: (0, i, 0)),
                  pl.BlockSpec((2, _BM), lambda i: (0, i)),
                  pl.BlockSpec((256,), lambda i: (0,)),
                  pl.BlockSpec((256, 128), lambda i: (0, 0))],
        out_specs=pl.BlockSpec((_BM, 128), lambda i: (i, 0)),
        out_shape=jax.ShapeDtypeStruct((NPAD, 128), jnp.float32),
    )(z1, deg2, b1, w2)


def _finalize(z2, xws2, deg2, b2):
    def body(z_ref, t_ref, deg_ref, b2_ref, o_ref):
        dinv = _dinv_of(deg_ref)
        b = b2_ref[...]
        o_ref[...] = (z_ref[0] + z_ref[1] - t_ref[...]) * dinv + b[None, :]

    return pl.pallas_call(
        body,
        grid=(NPAD // _BM,),
        in_specs=[pl.BlockSpec((2, _BM, 128), lambda i: (0, i, 0)),
                  pl.BlockSpec((_BM, 128), lambda i: (i, 0)),
                  pl.BlockSpec((2, _BM), lambda i: (0, i)),
                  pl.BlockSpec((128,), lambda i: (0,))],
        out_specs=pl.BlockSpec((_BM, 128), lambda i: (i, 0)),
        out_shape=jax.ShapeDtypeStruct((NPAD, 128), jnp.float32),
    )(z2, xws2, deg2, b2)


def kernel(x, adj_t, W1, b1, W2, b2):
    src = adj_t[0].astype(jnp.int32)
    dst = adj_t[1].astype(jnp.int32)
    pad = N + (jnp.arange(EPAD - E, dtype=jnp.int32) % (NPAD - N))
    srcp = jnp.concatenate([src, pad]).reshape(NCHUNK, CH)
    dstp = jnp.concatenate([dst, pad]).reshape(NCHUNK, CH)
    src3 = srcp.reshape(NSUB, CPS1, CH)
    dst3 = dstp.reshape(NSUB, CPS1, CH)
    src4 = srcp.reshape(NCORE, NSUB, CPS2, CH)
    dst4 = dstp.reshape(NCORE, NSUB, CPS2, CH)
    x_pad = jnp.pad(x, ((0, NPAD - N), (0, 0)))

    deg2 = _deg_kernel(dstp)
    xw1 = _matmul(x_pad, W1)
    xws1 = _scale_halves(xw1, deg2)
    z1 = _prop1(xws1, src3, dst3)
    xws2 = _layer2(z1, deg2, b1, W2)
    z2 = _prop2(xws2, src4, dst4)
    out = _finalize(z2, xws2, deg2, b2)
    return out[:N]


# speedup vs baseline: 27.4423x; 1.4986x over previous
import functools

import jax
import jax.numpy as jnp
from jax import lax
from jax.experimental import pallas as pl
from jax.experimental.pallas import tpu as pltpu
from jax.experimental.pallas import tpu_sc as plsc

N = 10000
NPAD = 10240
E = 320000
CH = 128
NCHUNK = 2560
EPAD = NCHUNK * CH
NSUB = 16
NCORE = 2
CPW = NCHUNK // (NSUB * NCORE)
CPS1 = NCHUNK // NSUB
CPS2 = NCHUNK // (NSUB * NCORE)
G = 40
STRIPE = NPAD // NSUB

_mesh = plsc.VectorSubcoreMesh(core_axis_name="c", subcore_axis_name="s")


@functools.partial(
    pl.kernel,
    out_type=jax.ShapeDtypeStruct((NCORE, NPAD), jnp.float32),
    mesh=_mesh,
    scratch_types=[
        pltpu.VMEM((CPW, CH), jnp.int32),
        pltpu.VMEM((CH,), jnp.float32),
        pltpu.VMEM((STRIPE,), jnp.float32),
        pltpu.VMEM_SHARED((NPAD,), jnp.float32),
    ],
)
def _deg_kernel(dst_hbm, deg_hbm, idx_v, ones_v, zeros_v, deg_sh):
    c = lax.axis_index("c")
    s = lax.axis_index("s")
    w = c * NSUB + s

    @pl.loop(0, CH, step=16)
    def _(i):
        ones_v[pl.ds(i, 16)] = jnp.ones((16,), jnp.float32)

    @pl.loop(0, STRIPE, step=16)
    def _(i):
        zeros_v[pl.ds(i, 16)] = jnp.zeros((16,), jnp.float32)

    pltpu.sync_copy(zeros_v, deg_sh.at[pl.ds(s * STRIPE, STRIPE)])
    pltpu.sync_copy(dst_hbm.at[pl.ds(w * CPW, CPW)], idx_v)
    plsc.subcore_barrier()

    @pl.loop(0, CPW)
    def _(j):
        pltpu.sync_copy(ones_v, deg_sh.at[idx_v.at[j]], add=True)

    plsc.subcore_barrier()
    pltpu.sync_copy(deg_sh.at[pl.ds(s * STRIPE, STRIPE)],
                    deg_hbm.at[c, pl.ds(s * STRIPE, STRIPE)])


def _make_prop(per_core_table):
    tshape = (NCORE, NPAD, 128) if per_core_table else (NPAD, 128)
    ishape = (NSUB, CPS1, CH) if per_core_table else (NCORE, NSUB, CPS2, CH)
    cps = CPS1 if per_core_table else CPS2

    @functools.partial(
        pl.kernel,
        out_type=jax.ShapeDtypeStruct((NCORE, NPAD, 128), jnp.float32),
        mesh=_mesh,
        scratch_types=[
            pltpu.VMEM((G, CH), jnp.int32),
            pltpu.VMEM((G, CH), jnp.int32),
            pltpu.VMEM((CH, 128), jnp.float32),
            pltpu.VMEM((CH, 128), jnp.float32),
            pltpu.VMEM_SHARED((NPAD, 128), jnp.float32),
            pltpu.SemaphoreType.DMA,
            pltpu.SemaphoreType.DMA,
            pltpu.SemaphoreType.DMA,
            pltpu.SemaphoreType.DMA,
        ],
    )
    def _prop(table_hbm, src_hbm, dst_hbm, z_hbm, src_v, dst_v,
              rows0, rows1, z_sh, gsem0, gsem1, ssem0, ssem1):
        c = lax.axis_index("c")
        s = lax.axis_index("s")
        table = table_hbm.at[c] if per_core_table else table_hbm
        slab = (lambda r, g: r.at[s, pl.ds(g, G)]) if per_core_table else (
            lambda r, g: r.at[c, s, pl.ds(g, G)])

        def gather(j, rows, sem):
            return pltpu.make_async_copy(table.at[src_v.at[j]], rows, sem)

        def scatter(j, rows, sem):
            return pltpu.make_async_copy(rows, z_sh.at[dst_v.at[j]], sem)

        pltpu.sync_copy(table.at[pl.ds(s * STRIPE, STRIPE)],
                        z_sh.at[pl.ds(s * STRIPE, STRIPE)])
        plsc.subcore_barrier()

        @pl.loop(0, cps, step=G)
        def _(g):
            pltpu.sync_copy(slab(src_hbm, g), src_v)
            pltpu.sync_copy(slab(dst_hbm, g), dst_v)
            gather(0, rows0, gsem0).start()

            @pl.loop(0, G, step=2)
            def _(j):
                @pl.when(j > 0)
                def _():
                    scatter(j, rows1, ssem1).wait()
                gather(j + 1, rows1, gsem1).start()
                gather(j, rows0, gsem0).wait()
                scatter(j, rows0, ssem0).start(add=True)
                scatter(j, rows0, ssem0).wait()

                @pl.when(j + 2 < G)
                def _():
                    gather(j + 2, rows0, gsem0).start()
                gather(j + 1, rows1, gsem1).wait()
                scatter(j + 1, rows1, ssem1).start(add=True)

            scatter(0, rows1, ssem1).wait()

        plsc.subcore_barrier()
        pltpu.sync_copy(z_sh.at[pl.ds(s * STRIPE, STRIPE)],
                        z_hbm.at[c, pl.ds(s * STRIPE, STRIPE)])

    return _prop


_prop1 = _make_prop(True)
_prop2 = _make_prop(False)


_BM = 1280


def _dot(a, b):
    return jax.lax.dot_general(a, b, (((1,), (0,)), ((), ())),
                               precision=lax.Precision.HIGHEST,
                               preferred_element_type=jnp.float32)


def _matmul(x, w):
    m, k = x.shape
    _, n = w.shape

    def body(x_ref, w_ref, o_ref):
        o_ref[...] = _dot(x_ref[...], w_ref[...])

    return pl.pallas_call(
        body,
        grid=(m // _BM,),
        in_specs=[pl.BlockSpec((_BM, k), lambda i: (i, 0)),
                  pl.BlockSpec((k, n), lambda i: (0, 0))],
        out_specs=pl.BlockSpec((_BM, n), lambda i: (i, 0)),
        out_shape=jax.ShapeDtypeStruct((m, n), jnp.float32),
    )(x, w)


def _dinv_of(deg_ref):
    d = deg_ref[...]
    return lax.rsqrt(1.0 + d[0] + d[1])[:, None]


def _scale_halves(xw, deg2):
    def body(xw_ref, deg_ref, o_ref):
        dinv = _dinv_of(deg_ref)
        xwb = xw_ref[...]
        o_ref[0] = xwb[:, :128] * dinv
        o_ref[1] = xwb[:, 128:] * dinv

    return pl.pallas_call(
        body,
        grid=(NPAD // _BM,),
        in_specs=[pl.BlockSpec((_BM, 256), lambda i: (i, 0)),
                  pl.BlockSpec((2, _BM), lambda i: (0, i))],
        out_specs=pl.BlockSpec((2, _BM, 128), lambda i: (0, i, 0)),
        out_shape=jax.ShapeDtypeStruct((2, NPAD, 128), jnp.float32),
    )(xw, deg2)


def _layer2(z1, deg2, b1, w2):
    def body(z_ref, deg_ref, b1_ref, w2_ref, o_ref):
        dinv = _dinv_of(deg_ref)
        b = b1_ref[...]
        h0 = jnp.maximum(z_ref[0] * dinv + b[None, :128], 0.0)
        h1 = jnp.maximum(z_ref[1] * dinv + b[None, 128:], 0.0)
        y = _dot(h0, w2_ref[:128]) + _dot(h1, w2_ref[128:])
        o_ref[...] = y * dinv

    return pl.pallas_call(
        body,
        grid=(NPAD // _BM,),
        in_specs=[pl.BlockSpec((2, _BM, 128), lambda i: (0, i, 0)),
                  pl.BlockSpec((2, _BM), lambda i: (0, i)),
                  pl.BlockSpec((256,), lambda i: (0,)),
                  pl.BlockSpec((256, 128), lambda i: (0, 0))],
        out_specs=pl.BlockSpec((_BM, 128), lambda i: (i, 0)),
        out_shape=jax.ShapeDtypeStruct((NPAD, 128), jnp.float32),
    )(z1, deg2, b1, w2)


def _finalize(z2, xws2, deg2, b2):
    def body(z_ref, t_ref, deg_ref, b2_ref, o_ref):
        dinv = _dinv_of(deg_ref)
        b = b2_ref[...]
        o_ref[...] = (z_ref[0] + z_ref[1] - t_ref[...]) * dinv + b[None, :]

    return pl.pallas_call(
        body,
        grid=(NPAD // _BM,),
        in_specs=[pl.BlockSpec((2, _BM, 128), lambda i: (0, i, 0)),
                  pl.BlockSpec((_BM, 128), lambda i: (i, 0)),
                  pl.BlockSpec((2, _BM), lambda i: (0, i)),
                  pl.BlockSpec((128,), lambda i: (0,))],
        out_specs=pl.BlockSpec((_BM, 128), lambda i: (i, 0)),
        out_shape=jax.ShapeDtypeStruct((NPAD, 128), jnp.float32),
    )(z2, xws2, deg2, b2)


def kernel(x, adj_t, W1, b1, W2, b2):
    src = adj_t[0].astype(jnp.int32)
    dst = adj_t[1].astype(jnp.int32)
    pad = N + (jnp.arange(EPAD - E, dtype=jnp.int32) % (NPAD - N))
    srcp = jnp.concatenate([src, pad]).reshape(NCHUNK, CH)
    dstp = jnp.concatenate([dst, pad]).reshape(NCHUNK, CH)
    src3 = srcp.reshape(NSUB, CPS1, CH)
    dst3 = dstp.reshape(NSUB, CPS1, CH)
    src4 = srcp.reshape(NCORE, NSUB, CPS2, CH)
    dst4 = dstp.reshape(NCORE, NSUB, CPS2, CH)
    x_pad = jnp.pad(x, ((0, NPAD - N), (0, 0)))

    deg2 = _deg_kernel(dstp)
    xw1 = _matmul(x_pad, W1)
    xws1 = _scale_halves(xw1, deg2)
    z1 = _prop1(xws1, src3, dst3)
    xws2 = _layer2(z1, deg2, b1, W2)
    z2 = _prop2(xws2, src4, dst4)
    out = _finalize(z2, xws2, deg2, b2)
    return out[:N]


# speedup vs baseline: 27.8430x; 1.0146x over previous
import functools

import jax
import jax.numpy as jnp
from jax import lax
from jax.experimental import pallas as pl
from jax.experimental.pallas import tpu as pltpu
from jax.experimental.pallas import tpu_sc as plsc

N = 10000
NPAD = 10240
E = 320000
CH = 128
NCHUNK = 2560
EPAD = NCHUNK * CH
NSUB = 16
NCORE = 2
CPW = NCHUNK // (NSUB * NCORE)
CPS1 = NCHUNK // NSUB
CPS2 = NCHUNK // (NSUB * NCORE)
G = 40
STRIPE = NPAD // NSUB

_mesh = plsc.VectorSubcoreMesh(core_axis_name="c", subcore_axis_name="s")


@functools.partial(
    pl.kernel,
    out_type=jax.ShapeDtypeStruct((NCORE, NPAD), jnp.float32),
    mesh=_mesh,
    scratch_types=[
        pltpu.VMEM((CPW, CH), jnp.int32),
        pltpu.VMEM((CH,), jnp.float32),
        pltpu.VMEM((STRIPE,), jnp.float32),
        pltpu.VMEM_SHARED((NPAD,), jnp.float32),
    ],
)
def _deg_kernel(dst_hbm, deg_hbm, idx_v, ones_v, zeros_v, deg_sh):
    c = lax.axis_index("c")
    s = lax.axis_index("s")
    w = c * NSUB + s

    @pl.loop(0, CH, step=16)
    def _(i):
        ones_v[pl.ds(i, 16)] = jnp.ones((16,), jnp.float32)

    @pl.loop(0, STRIPE, step=16)
    def _(i):
        zeros_v[pl.ds(i, 16)] = jnp.zeros((16,), jnp.float32)

    pltpu.sync_copy(zeros_v, deg_sh.at[pl.ds(s * STRIPE, STRIPE)])
    pltpu.sync_copy(dst_hbm.at[pl.ds(w * CPW, CPW)], idx_v)
    plsc.subcore_barrier()

    @pl.loop(0, CPW)
    def _(j):
        pltpu.sync_copy(ones_v, deg_sh.at[idx_v.at[j]], add=True)

    plsc.subcore_barrier()
    pltpu.sync_copy(deg_sh.at[pl.ds(s * STRIPE, STRIPE)],
                    deg_hbm.at[c, pl.ds(s * STRIPE, STRIPE)])


def _make_prop(per_core_table):
    tshape = (NCORE, NPAD, 128) if per_core_table else (NPAD, 128)
    ishape = (NSUB, CPS1, CH) if per_core_table else (NCORE, NSUB, CPS2, CH)
    cps = CPS1 if per_core_table else CPS2

    @functools.partial(
        pl.kernel,
        out_type=jax.ShapeDtypeStruct((NCORE, NPAD, 128), jnp.float32),
        mesh=_mesh,
        scratch_types=[
            pltpu.VMEM((G, CH), jnp.int32),
            pltpu.VMEM((G, CH), jnp.int32),
            pltpu.VMEM((CH, 128), jnp.float32),
            pltpu.VMEM((CH, 128), jnp.float32),
            pltpu.VMEM_SHARED((NPAD, 128), jnp.float32),
            pltpu.SemaphoreType.DMA,
            pltpu.SemaphoreType.DMA,
            pltpu.SemaphoreType.DMA,
            pltpu.SemaphoreType.DMA,
        ],
    )
    def _prop(table_hbm, src_hbm, dst_hbm, z_hbm, src_v, dst_v,
              rows0, rows1, z_sh, gsem0, gsem1, ssem0, ssem1):
        c = lax.axis_index("c")
        s = lax.axis_index("s")
        table = table_hbm.at[c] if per_core_table else table_hbm
        slab = (lambda r, g: r.at[s, pl.ds(g, G)]) if per_core_table else (
            lambda r, g: r.at[c, s, pl.ds(g, G)])

        def gather(j, rows, sem):
            return pltpu.make_async_copy(table.at[src_v.at[j]], rows, sem)

        def scatter(j, rows, sem):
            return pltpu.make_async_copy(rows, z_sh.at[dst_v.at[j]], sem)

        pltpu.sync_copy(table.at[pl.ds(s * STRIPE, STRIPE)],
                        z_sh.at[pl.ds(s * STRIPE, STRIPE)])
        plsc.subcore_barrier()

        @pl.loop(0, cps, step=G)
        def _(g):
            ls = pltpu.make_async_copy(slab(src_hbm, g), src_v, gsem0)
            ld = pltpu.make_async_copy(slab(dst_hbm, g), dst_v, gsem1)
            ls.start()
            ld.start()
            ls.wait()
            ld.wait()
            gather(0, rows0, gsem0).start()

            @pl.loop(0, G, step=2)
            def _(j):
                @pl.when(j > 0)
                def _():
                    scatter(j, rows1, ssem1).wait()
                gather(j + 1, rows1, gsem1).start()
                gather(j, rows0, gsem0).wait()
                scatter(j, rows0, ssem0).start(add=True)
                scatter(j, rows0, ssem0).wait()

                @pl.when(j + 2 < G)
                def _():
                    gather(j + 2, rows0, gsem0).start()
                gather(j + 1, rows1, gsem1).wait()
                scatter(j + 1, rows1, ssem1).start(add=True)

            scatter(0, rows1, ssem1).wait()

        plsc.subcore_barrier()
        pltpu.sync_copy(z_sh.at[pl.ds(s * STRIPE, STRIPE)],
                        z_hbm.at[c, pl.ds(s * STRIPE, STRIPE)])

    return _prop


_prop1 = _make_prop(True)
_prop2 = _make_prop(False)


_BM = 1280


def _dot(a, b):
    return jax.lax.dot_general(a, b, (((1,), (0,)), ((), ())),
                               precision=lax.Precision.HIGHEST,
                               preferred_element_type=jnp.float32)


def _dinv_of(deg_ref):
    d = deg_ref[...]
    return lax.rsqrt(1.0 + d[0] + d[1])[:, None]


def _mm_scale(x, w, deg2):
    def body(x_ref, w_ref, deg_ref, o_ref):
        dinv = _dinv_of(deg_ref)
        y = _dot(x_ref[...], w_ref[...])
        o_ref[0] = y[:, :128] * dinv
        o_ref[1] = y[:, 128:] * dinv

    return pl.pallas_call(
        body,
        grid=(NPAD // _BM,),
        in_specs=[pl.BlockSpec((_BM, 128), lambda i: (i, 0)),
                  pl.BlockSpec((128, 256), lambda i: (0, 0)),
                  pl.BlockSpec((2, _BM), lambda i: (0, i))],
        out_specs=pl.BlockSpec((2, _BM, 128), lambda i: (0, i, 0)),
        out_shape=jax.ShapeDtypeStruct((2, NPAD, 128), jnp.float32),
    )(x, w, deg2)


def _layer2(z1, deg2, b1, w2):
    def body(z_ref, deg_ref, b1_ref, w2_ref, o_ref):
        dinv = _dinv_of(deg_ref)
        b = b1_ref[...]
        h0 = jnp.maximum(z_ref[0] * dinv + b[None, :128], 0.0)
        h1 = jnp.maximum(z_ref[1] * dinv + b[None, 128:], 0.0)
        y = _dot(h0, w2_ref[:128]) + _dot(h1, w2_ref[128:])
        o_ref[...] = y * dinv

    return pl.pallas_call(
        body,
        grid=(NPAD // _BM,),
        in_specs=[pl.BlockSpec((2, _BM, 128), lambda i: (0, i, 0)),
                  pl.BlockSpec((2, _BM), lambda i: (0, i)),
                  pl.BlockSpec((256,), lambda i: (0,)),
                  pl.BlockSpec((256, 128), lambda i: (0, 0))],
        out_specs=pl.BlockSpec((_BM, 128), lambda i: (i, 0)),
        out_shape=jax.ShapeDtypeStruct((NPAD, 128), jnp.float32),
    )(z1, deg2, b1, w2)


def _finalize(z2, xws2, deg2, b2):
    def body(z_ref, t_ref, deg_ref, b2_ref, o_ref):
        dinv = _dinv_of(deg_ref)
        b = b2_ref[...]
        o_ref[...] = (z_ref[0] + z_ref[1] - t_ref[...]) * dinv + b[None, :]

    return pl.pallas_call(
        body,
        grid=(NPAD // _BM,),
        in_specs=[pl.BlockSpec((2, _BM, 128), lambda i: (0, i, 0)),
                  pl.BlockSpec((_BM, 128), lambda i: (i, 0)),
                  pl.BlockSpec((2, _BM), lambda i: (0, i)),
                  pl.BlockSpec((128,), lambda i: (0,))],
        out_specs=pl.BlockSpec((_BM, 128), lambda i: (i, 0)),
        out_shape=jax.ShapeDtypeStruct((NPAD, 128), jnp.float32),
    )(z2, xws2, deg2, b2)


def kernel(x, adj_t, W1, b1, W2, b2):
    src = adj_t[0].astype(jnp.int32)
    dst = adj_t[1].astype(jnp.int32)
    pad = N + (jnp.arange(EPAD - E, dtype=jnp.int32) % (NPAD - N))
    srcp = jnp.concatenate([src, pad]).reshape(NCHUNK, CH)
    dstp = jnp.concatenate([dst, pad]).reshape(NCHUNK, CH)
    src3 = srcp.reshape(NSUB, CPS1, CH)
    dst3 = dstp.reshape(NSUB, CPS1, CH)
    src4 = srcp.reshape(NCORE, NSUB, CPS2, CH)
    dst4 = dstp.reshape(NCORE, NSUB, CPS2, CH)
    x_pad = jnp.pad(x, ((0, NPAD - N), (0, 0)))

    deg2 = _deg_kernel(dstp)
    xws1 = _mm_scale(x_pad, W1, deg2)
    z1 = _prop1(xws1, src3, dst3)
    xws2 = _layer2(z1, deg2, b1, W2)
    z2 = _prop2(xws2, src4, dst4)
    out = _finalize(z2, xws2, deg2, b2)
    return out[:N]
